# Initial kernel scaffold; baseline (speedup 1.0000x reference)
import jax
import jax.numpy as jnp
from jax.experimental import pallas as pl


def kernel(x, edge_index, W):
    raise NotImplementedError("write your pallas kernel here")


# speedup vs baseline: 8.4306x; 8.4306x over previous
import jax
import jax.numpy as jnp
from jax import lax
from jax.experimental import pallas as pl
from jax.experimental.pallas import tpu as pltpu
from jax.experimental.pallas import tpu_sc as plsc

NC = 2
NS = 16
NW = NC * NS
CH = 80
CPT = 128
RB = 1024


def _deg_body(dst2_hbm, deg_hbm, deg_sh, didx, zeros_v, ones_v):
    c = lax.axis_index("c")
    s = lax.axis_index("s")
    npad = deg_sh.shape[0]
    rpt = npad // NS
    for j in range(rpt // 16):
        zeros_v[pl.ds(j * 16, 16)] = jnp.zeros((16,), jnp.float32)
    for j in range(CH // 16):
        ones_v[pl.ds(j * 16, 16)] = jnp.full((16,), 1.0, jnp.float32)
    pltpu.sync_copy(zeros_v, deg_sh.at[pl.ds(s * rpt, rpt)])
    rbase = (c * NS + s) * CPT
    pltpu.sync_copy(dst2_hbm.at[pl.ds(rbase, CPT)], didx)
    plsc.subcore_barrier()

    def chunk(i, carry):
        pltpu.sync_copy(ones_v, deg_sh.at[didx.at[i]], add=True)
        return carry

    lax.fori_loop(0, CPT, chunk, 0)
    plsc.subcore_barrier()
    pltpu.sync_copy(deg_sh.at[pl.ds(s * rpt, rpt)],
                    deg_hbm.at[c, pl.ds(s * rpt, rpt)])


def _hop_body(y_hbm, src2_hbm, dst2_hbm, out_hbm, acc_sh, sidx, didx, rows,
              sem):
    c = lax.axis_index("c")
    s = lax.axis_index("s")
    npad = y_hbm.shape[0]
    rpt = npad // NS
    pltpu.sync_copy(y_hbm.at[pl.ds(s * rpt, rpt)],
                    acc_sh.at[pl.ds(s * rpt, rpt)])
    rbase = (c * NS + s) * CPT
    pltpu.sync_copy(src2_hbm.at[pl.ds(rbase, CPT)], sidx)
    pltpu.sync_copy(dst2_hbm.at[pl.ds(rbase, CPT)], didx)
    plsc.subcore_barrier()

    def chunk(i, carry):
        pltpu.async_copy(y_hbm.at[sidx.at[i]], rows, sem).wait()
        pltpu.sync_copy(rows, acc_sh.at[didx.at[i]], add=True)
        return carry

    lax.fori_loop(0, CPT, chunk, 0)
    plsc.subcore_barrier()
    pltpu.sync_copy(acc_sh.at[pl.ds(s * rpt, rpt)],
                    out_hbm.at[c, pl.ds(s * rpt, rpt)])


def _scale0_body(degt_ref, x_ref, y_ref):
    d = degt_ref[:, 0:1] + degt_ref[:, 1:2] + 1.0
    y_ref[...] = x_ref[...] * lax.rsqrt(d)


def _scale_mid_body(degt_ref, p_ref, y0_ref, y1_ref):
    d = degt_ref[:, 0:1] + degt_ref[:, 1:2] + 1.0
    a = p_ref[0, :, :] + p_ref[1, :, :] - y0_ref[...]
    y1_ref[...] = a / d


def _final_body(degt_ref, q_ref, y1_ref, w_ref, out_ref):
    d = degt_ref[:, 0:1] + degt_ref[:, 1:2] + 1.0
    h = (q_ref[0, :, :] + q_ref[1, :, :] - y1_ref[...]) * lax.rsqrt(d)
    out_ref[...] = lax.dot_general(h, w_ref[...], (((1,), (1,)), ((), ())),
                                   preferred_element_type=jnp.float32)


def kernel(x, edge_index, W):
    n, d = x.shape
    e = edge_index.shape[1]
    npad = ((n + NS * 16 - 1) // (NS * 16)) * (NS * 16)
    epad = NW * CPT * CH
    assert epad >= e and npad % RB == 0
    xp = jnp.pad(x, ((0, npad - n), (0, 0)))
    src2 = jnp.pad(edge_index[0], (0, epad - e)).reshape(epad // CH, CH)
    dst2 = jnp.pad(edge_index[1], (0, epad - e),
                   constant_values=n).reshape(epad // CH, CH)
    mesh = plsc.VectorSubcoreMesh(core_axis_name="c", subcore_axis_name="s")

    deg = pl.kernel(
        _deg_body,
        out_type=jax.ShapeDtypeStruct((NC, npad), jnp.float32),
        mesh=mesh,
        scratch_types=[
            pltpu.VMEM_SHARED((npad,), jnp.float32),
            pltpu.VMEM((CPT, CH), jnp.int32),
            pltpu.VMEM((npad // NS,), jnp.float32),
            pltpu.VMEM((CH,), jnp.float32),
        ],
    )(dst2)
    degt = deg.T

    hop = pl.kernel(
        _hop_body,
        out_type=jax.ShapeDtypeStruct((NC, npad, d), jnp.float32),
        mesh=mesh,
        scratch_types=[
            pltpu.VMEM_SHARED((npad, d), jnp.float32),
            pltpu.VMEM((CPT, CH), jnp.int32),
            pltpu.VMEM((CPT, CH), jnp.int32),
            pltpu.VMEM((CH, d), jnp.float32),
            pltpu.SemaphoreType.DMA,
        ],
    )

    nb = npad // RB
    row = lambda i: (i, 0)
    full2 = pl.BlockSpec((RB, d), row)
    degb = pl.BlockSpec((RB, NC), row)
    pairb = pl.BlockSpec((NC, RB, d), lambda i: (0, i, 0))

    y0 = pl.pallas_call(
        _scale0_body,
        grid=(nb,),
        in_specs=[degb, full2],
        out_specs=full2,
        out_shape=jax.ShapeDtypeStruct((npad, d), jnp.float32),
    )(degt, xp)

    p = hop(y0, src2, dst2)

    y1 = pl.pallas_call(
        _scale_mid_body,
        grid=(nb,),
        in_specs=[degb, pairb, full2],
        out_specs=full2,
        out_shape=jax.ShapeDtypeStruct((npad, d), jnp.float32),
    )(degt, p, y0)

    q = hop(y1, src2, dst2)

    out = pl.pallas_call(
        _final_body,
        grid=(nb,),
        in_specs=[degb, pairb, full2,
                  pl.BlockSpec((d, d), lambda i: (0, 0))],
        out_specs=full2,
        out_shape=jax.ShapeDtypeStruct((npad, d), jnp.float32),
    )(degt, q, y1, W)
    return out[:n]


# speedup vs baseline: 21.9994x; 2.6095x over previous
import jax
import jax.numpy as jnp
from jax import lax
from jax.experimental import pallas as pl
from jax.experimental.pallas import tpu as pltpu
from jax.experimental.pallas import tpu_sc as plsc

NC = 2
NS = 16
NW = NC * NS
CH = 64
CPT = 160
SB = 2
NP = 10112
NPD = 10240
RB = 1264


def _deg_body(dst2_hbm, deg_hbm, deg_sh, didx, zeros_v, ones_v):
    c = lax.axis_index("c")
    s = lax.axis_index("s")
    npad = deg_sh.shape[0]
    rpt = npad // NS
    for j in range((rpt + 15) // 16):
        zeros_v[pl.ds(min(j * 16, rpt - 16), 16)] = jnp.zeros((16,),
                                                              jnp.float32)
    for j in range(CH // 16):
        ones_v[pl.ds(j * 16, 16)] = jnp.full((16,), 1.0, jnp.float32)
    pltpu.sync_copy(zeros_v, deg_sh.at[pl.ds(s * rpt, rpt)])
    rbase = (c * NS + s) * CPT
    pltpu.sync_copy(dst2_hbm.at[pl.ds(rbase, CPT)], didx)
    plsc.subcore_barrier()

    def chunk(i, carry):
        pltpu.sync_copy(ones_v, deg_sh.at[didx.at[i]], add=True)
        return carry

    lax.fori_loop(0, CPT, chunk, 0)
    plsc.subcore_barrier()
    pltpu.sync_copy(deg_sh.at[pl.ds(s * rpt, rpt)],
                    deg_hbm.at[c, 0, pl.ds(s * rpt, rpt)])


def _hop_body(y_hbm, src2_hbm, dst2_hbm, out_hbm, acc_sh, sidx, didx, rows0,
              rows1, sem0, sem1):
    c = lax.axis_index("c")
    s = lax.axis_index("s")
    npad = y_hbm.shape[0]
    rpt = npad // NS
    pltpu.sync_copy(y_hbm.at[pl.ds(s * rpt, rpt)],
                    acc_sh.at[pl.ds(s * rpt, rpt)])
    rbase = (c * NS + s) * CPT
    plsc.subcore_barrier()

    def gather(i, buf, sem):
        pltpu.async_copy(y_hbm.at[sidx.at[i]], buf, sem)

    def gwait(buf, sem):
        pltpu.make_async_copy(y_hbm.at[sidx.at[0]], buf, sem).wait()

    def scatter(i, buf):
        pltpu.sync_copy(buf, acc_sh.at[didx.at[i]], add=True)

    cps = CPT // SB
    for h in range(SB):
        pltpu.sync_copy(src2_hbm.at[pl.ds(rbase + h * cps, cps)], sidx)
        pltpu.sync_copy(dst2_hbm.at[pl.ds(rbase + h * cps, cps)], didx)
        gather(0, rows0, sem0)

        def pair(j, carry):
            i0 = 2 * j
            gwait(rows0, sem0)
            gather(i0 + 1, rows1, sem1)
            scatter(i0, rows0)
            gwait(rows1, sem1)
            gather(i0 + 2, rows0, sem0)
            scatter(i0 + 1, rows1)
            return carry

        lax.fori_loop(0, cps // 2 - 1, pair, 0)
        gwait(rows0, sem0)
        gather(cps - 1, rows1, sem1)
        scatter(cps - 2, rows0)
        gwait(rows1, sem1)
        scatter(cps - 1, rows1)
    plsc.subcore_barrier()
    pltpu.sync_copy(acc_sh.at[pl.ds(s * rpt, rpt)],
                    out_hbm.at[c, pl.ds(s * rpt, rpt)])


def _scale0_body(degt_ref, x_ref, y_ref):
    d = degt_ref[:, 0:1] + degt_ref[:, 1:2] + 1.0
    y_ref[...] = x_ref[...] * lax.rsqrt(d)


def _scale_mid_body(degt_ref, p_ref, y0_ref, y1_ref):
    d = degt_ref[:, 0:1] + degt_ref[:, 1:2] + 1.0
    a = p_ref[0, :, :] + p_ref[1, :, :] - y0_ref[...]
    y1_ref[...] = a / d


def _final_body(degt_ref, q_ref, y1_ref, w_ref, out_ref):
    d = degt_ref[:, 0:1] + degt_ref[:, 1:2] + 1.0
    h = (q_ref[0, :, :] + q_ref[1, :, :] - y1_ref[...]) * lax.rsqrt(d)
    out_ref[...] = lax.dot_general(h, w_ref[...], (((1,), (1,)), ((), ())),
                                   preferred_element_type=jnp.float32)


def kernel(x, edge_index, W):
    n, d = x.shape
    e = edge_index.shape[1]
    npad = NP
    epad = NW * CPT * CH
    assert epad >= e and npad >= n and npad % RB == 0 and (npad // NS) % 8 == 0
    xp = jnp.pad(x, ((0, npad - n), (0, 0)))
    pad_src = jnp.arange(epad - e, dtype=jnp.int32) % n
    pad_dst = n + jnp.arange(epad - e, dtype=jnp.int32) % (npad - n)
    src2 = jnp.concatenate([edge_index[0], pad_src]).reshape(epad // CH, CH)
    dst2 = jnp.concatenate([edge_index[1], pad_dst]).reshape(epad // CH, CH)
    mesh = plsc.VectorSubcoreMesh(core_axis_name="c", subcore_axis_name="s")

    deg = pl.kernel(
        _deg_body,
        out_type=jax.ShapeDtypeStruct((NC, 1, NPD), jnp.float32),
        mesh=mesh,
        scratch_types=[
            pltpu.VMEM_SHARED((NPD,), jnp.float32),
            pltpu.VMEM((CPT, CH), jnp.int32),
            pltpu.VMEM((NPD // NS,), jnp.float32),
            pltpu.VMEM((CH,), jnp.float32),
        ],
    )(dst2)
    degt = deg[:, 0, :npad].T

    hop = pl.kernel(
        _hop_body,
        out_type=jax.ShapeDtypeStruct((NC, npad, d), jnp.float32),
        mesh=mesh,
        scratch_types=[
            pltpu.VMEM_SHARED((npad, d), jnp.float32),
            pltpu.VMEM((CPT // SB, CH), jnp.int32),
            pltpu.VMEM((CPT // SB, CH), jnp.int32),
            pltpu.VMEM((CH, d), jnp.float32),
            pltpu.VMEM((CH, d), jnp.float32),
            pltpu.SemaphoreType.DMA,
            pltpu.SemaphoreType.DMA,
        ],
    )

    nb = npad // RB
    row = lambda i: (i, 0)
    full2 = pl.BlockSpec((RB, d), row)
    degb = pl.BlockSpec((RB, NC), row)
    pairb = pl.BlockSpec((NC, RB, d), lambda i: (0, i, 0))

    y0 = pl.pallas_call(
        _scale0_body,
        grid=(nb,),
        in_specs=[degb, full2],
        out_specs=full2,
        out_shape=jax.ShapeDtypeStruct((npad, d), jnp.float32),
    )(degt, xp)

    p = hop(y0, src2, dst2)

    y1 = pl.pallas_call(
        _scale_mid_body,
        grid=(nb,),
        in_specs=[degb, pairb, full2],
        out_specs=full2,
        out_shape=jax.ShapeDtypeStruct((npad, d), jnp.float32),
    )(degt, p, y0)

    q = hop(y1, src2, dst2)

    out = pl.pallas_call(
        _final_body,
        grid=(nb,),
        in_specs=[degb, pairb, full2,
                  pl.BlockSpec((d, d), lambda i: (0, 0))],
        out_specs=full2,
        out_shape=jax.ShapeDtypeStruct((npad, d), jnp.float32),
    )(degt, q, y1, W)
    return out[:n]


# speedup vs baseline: 22.6428x; 1.0292x over previous
import jax
import jax.numpy as jnp
from jax import lax
from jax.experimental import pallas as pl
from jax.experimental.pallas import tpu as pltpu
from jax.experimental.pallas import tpu_sc as plsc

NC = 2
NS = 16
NW = NC * NS
CH = 32
CPT = 320
SB = 8
NP = 10112
NPD = 10240
RB = 1264


def _deg_body(dst2_hbm, deg_hbm, deg_sh, didx, zeros_v, ones_v):
    c = lax.axis_index("c")
    s = lax.axis_index("s")
    npad = deg_sh.shape[0]
    rpt = npad // NS
    for j in range((rpt + 15) // 16):
        zeros_v[pl.ds(min(j * 16, rpt - 16), 16)] = jnp.zeros((16,),
                                                              jnp.float32)
    for j in range(CH // 16):
        ones_v[pl.ds(j * 16, 16)] = jnp.full((16,), 1.0, jnp.float32)
    pltpu.sync_copy(zeros_v, deg_sh.at[pl.ds(s * rpt, rpt)])
    rbase = (c * NS + s) * CPT
    pltpu.sync_copy(dst2_hbm.at[pl.ds(rbase, CPT)], didx)
    plsc.subcore_barrier()

    def chunk(i, carry):
        pltpu.sync_copy(ones_v, deg_sh.at[didx.at[i]], add=True)
        return carry

    lax.fori_loop(0, CPT, chunk, 0)
    plsc.subcore_barrier()
    pltpu.sync_copy(deg_sh.at[pl.ds(s * rpt, rpt)],
                    deg_hbm.at[c, 0, pl.ds(s * rpt, rpt)])


def _hop_body(y_hbm, src2_hbm, dst2_hbm, out_hbm, acc_sh, sidx, didx,
              b0, b1, b2, b3, g0, g1, g2, g3):
    c = lax.axis_index("c")
    s = lax.axis_index("s")
    npad = y_hbm.shape[0]
    rpt = npad // NS
    pltpu.sync_copy(y_hbm.at[pl.ds(s * rpt, rpt)],
                    acc_sh.at[pl.ds(s * rpt, rpt)])
    rbase = (c * NS + s) * CPT
    plsc.subcore_barrier()

    bufs = (b0, b1, b2, b3)
    gsem = (g0, g1, g2, g3)

    def gather(i, k):
        pltpu.async_copy(y_hbm.at[sidx.at[i]], bufs[k], gsem[k])

    def gwait(k):
        pltpu.make_async_copy(y_hbm.at[sidx.at[0]], bufs[k], gsem[k]).wait()

    def scatter(i, k):
        pltpu.sync_copy(bufs[k], acc_sh.at[didx.at[i]], add=True)

    def step(i, k, do_gather=True):
        gwait(k)
        if do_gather:
            gather(i + 2, (k + 2) % 4)
        scatter(i, k)

    cpb = CPT // SB
    for h in range(SB):
        pltpu.sync_copy(src2_hbm.at[pl.ds(rbase + h * cpb, cpb)], sidx)
        pltpu.sync_copy(dst2_hbm.at[pl.ds(rbase + h * cpb, cpb)], didx)
        gather(0, 0)
        gather(1, 1)
        step(0, 0)
        step(1, 1)

        def quad(j, carry):
            i0 = 4 * j + 2
            step(i0, 2)
            step(i0 + 1, 3)
            step(i0 + 2, 0)
            step(i0 + 3, 1)
            return carry

        lax.fori_loop(0, (cpb - 4) // 4, quad, 0)
        step(cpb - 2, 2, do_gather=False)
        step(cpb - 1, 3, do_gather=False)
    plsc.subcore_barrier()
    pltpu.sync_copy(acc_sh.at[pl.ds(s * rpt, rpt)],
                    out_hbm.at[c, pl.ds(s * rpt, rpt)])


def _scale0_body(degt_ref, x_ref, y_ref):
    d = degt_ref[:, 0:1] + degt_ref[:, 1:2] + 1.0
    y_ref[...] = x_ref[...] * lax.rsqrt(d)


def _scale_mid_body(degt_ref, p_ref, y0_ref, y1_ref):
    d = degt_ref[:, 0:1] + degt_ref[:, 1:2] + 1.0
    a = p_ref[0, :, :] + p_ref[1, :, :] - y0_ref[...]
    y1_ref[...] = a / d


def _final_body(degt_ref, q_ref, y1_ref, w_ref, out_ref):
    d = degt_ref[:, 0:1] + degt_ref[:, 1:2] + 1.0
    h = (q_ref[0, :, :] + q_ref[1, :, :] - y1_ref[...]) * lax.rsqrt(d)
    out_ref[...] = lax.dot_general(h, w_ref[...], (((1,), (1,)), ((), ())),
                                   preferred_element_type=jnp.float32)


def kernel(x, edge_index, W):
    n, d = x.shape
    e = edge_index.shape[1]
    npad = NP
    epad = NW * CPT * CH
    assert epad >= e and npad >= n and npad % RB == 0 and (npad // NS) % 8 == 0
    xp = jnp.pad(x, ((0, npad - n), (0, 0)))
    pad_src = jnp.arange(epad - e, dtype=jnp.int32) % n
    pad_dst = n + jnp.arange(epad - e, dtype=jnp.int32) % (npad - n)
    src2 = jnp.concatenate([edge_index[0], pad_src]).reshape(epad // CH, CH)
    dst2 = jnp.concatenate([edge_index[1], pad_dst]).reshape(epad // CH, CH)
    mesh = plsc.VectorSubcoreMesh(core_axis_name="c", subcore_axis_name="s")

    deg = pl.kernel(
        _deg_body,
        out_type=jax.ShapeDtypeStruct((NC, 1, NPD), jnp.float32),
        mesh=mesh,
        scratch_types=[
            pltpu.VMEM_SHARED((NPD,), jnp.float32),
            pltpu.VMEM((CPT, CH), jnp.int32),
            pltpu.VMEM((NPD // NS,), jnp.float32),
            pltpu.VMEM((CH,), jnp.float32),
        ],
    )(dst2)
    degt = deg[:, 0, :npad].T

    hop = pl.kernel(
        _hop_body,
        out_type=jax.ShapeDtypeStruct((NC, npad, d), jnp.float32),
        mesh=mesh,
        scratch_types=[
            pltpu.VMEM_SHARED((npad, d), jnp.float32),
            pltpu.VMEM((CPT // SB, CH), jnp.int32),
            pltpu.VMEM((CPT // SB, CH), jnp.int32),
            pltpu.VMEM((CH, d), jnp.float32),
            pltpu.VMEM((CH, d), jnp.float32),
            pltpu.VMEM((CH, d), jnp.float32),
            pltpu.VMEM((CH, d), jnp.float32),
            pltpu.SemaphoreType.DMA,
            pltpu.SemaphoreType.DMA,
            pltpu.SemaphoreType.DMA,
            pltpu.SemaphoreType.DMA,
        ],
    )

    nb = npad // RB
    row = lambda i: (i, 0)
    full2 = pl.BlockSpec((RB, d), row)
    degb = pl.BlockSpec((RB, NC), row)
    pairb = pl.BlockSpec((NC, RB, d), lambda i: (0, i, 0))

    y0 = pl.pallas_call(
        _scale0_body,
        grid=(nb,),
        in_specs=[degb, full2],
        out_specs=full2,
        out_shape=jax.ShapeDtypeStruct((npad, d), jnp.float32),
    )(degt, xp)

    p = hop(y0, src2, dst2)

    y1 = pl.pallas_call(
        _scale_mid_body,
        grid=(nb,),
        in_specs=[degb, pairb, full2],
        out_specs=full2,
        out_shape=jax.ShapeDtypeStruct((npad, d), jnp.float32),
    )(degt, p, y0)

    q = hop(y1, src2, dst2)

    out = pl.pallas_call(
        _final_body,
        grid=(nb,),
        in_specs=[degb, pairb, full2,
                  pl.BlockSpec((d, d), lambda i: (0, 0))],
        out_specs=full2,
        out_shape=jax.ShapeDtypeStruct((npad, d), jnp.float32),
    )(degt, q, y1, W)
    return out[:n]


# speedup vs baseline: 31.3663x; 1.3853x over previous
import jax
import jax.numpy as jnp
from jax import lax
from jax.experimental import pallas as pl
from jax.experimental.pallas import tpu as pltpu
from jax.experimental.pallas import tpu_sc as plsc

NC = 2
NS = 16
NW = NC * NS
CH = 64
CPT = 160
SB = 4
CHD = 128
CPTD = 80
NP = 10112
NPD = 10240
RB = 1264


def _deg_body(dst2_hbm, deg_hbm, deg_sh, didx, zeros_v, ones_v):
    c = lax.axis_index("c")
    s = lax.axis_index("s")
    npad = deg_sh.shape[0]
    rpt = npad // NS
    for j in range((rpt + 15) // 16):
        zeros_v[pl.ds(min(j * 16, rpt - 16), 16)] = jnp.zeros((16,),
                                                              jnp.float32)
    for j in range(CHD // 16):
        ones_v[pl.ds(j * 16, 16)] = jnp.full((16,), 1.0, jnp.float32)
    pltpu.sync_copy(zeros_v, deg_sh.at[pl.ds(s * rpt, rpt)])
    rbase = (c * NS + s) * CPTD
    pltpu.sync_copy(dst2_hbm.at[pl.ds(rbase, CPTD)], didx)
    plsc.subcore_barrier()

    def chunk(i, carry):
        pltpu.sync_copy(ones_v, deg_sh.at[didx.at[i]], add=True)
        return carry

    lax.fori_loop(0, CPTD, chunk, 0)
    plsc.subcore_barrier()
    pltpu.sync_copy(deg_sh.at[pl.ds(s * rpt, rpt)],
                    deg_hbm.at[c, 0, pl.ds(s * rpt, rpt)])


def _hop_body(y_hbm, src2_hbm, dst2_hbm, out_hbm, acc_sh, sidx, didx,
              b0, b1, b2, b3, g0, g1, g2, g3):
    c = lax.axis_index("c")
    s = lax.axis_index("s")
    npad = y_hbm.shape[0]
    rpt = npad // NS
    pltpu.sync_copy(y_hbm.at[pl.ds(s * rpt, rpt)],
                    acc_sh.at[pl.ds(s * rpt, rpt)])
    rbase = (c * NS + s) * CPT
    plsc.subcore_barrier()

    bufs = (b0, b1, b2, b3)
    gsem = (g0, g1, g2, g3)

    def gather(i, k):
        pltpu.async_copy(y_hbm.at[sidx.at[i]], bufs[k], gsem[k])

    def gwait(k):
        pltpu.make_async_copy(y_hbm.at[sidx.at[0]], bufs[k], gsem[k]).wait()

    def scatter(i, k):
        pltpu.sync_copy(bufs[k], acc_sh.at[didx.at[i]], add=True)

    def step(i, k, do_gather=True):
        gwait(k)
        if do_gather:
            gather(i + 2, (k + 2) % 4)
        scatter(i, k)

    cpb = CPT // SB
    for h in range(SB):
        pltpu.sync_copy(src2_hbm.at[pl.ds(rbase + h * cpb, cpb)], sidx)
        pltpu.sync_copy(dst2_hbm.at[pl.ds(rbase + h * cpb, cpb)], didx)
        gather(0, 0)
        gather(1, 1)
        step(0, 0)
        step(1, 1)

        def quad(j, carry):
            i0 = 4 * j + 2
            step(i0, 2)
            step(i0 + 1, 3)
            step(i0 + 2, 0)
            step(i0 + 3, 1)
            return carry

        lax.fori_loop(0, (cpb - 4) // 4, quad, 0)
        step(cpb - 2, 2, do_gather=False)
        step(cpb - 1, 3, do_gather=False)
    plsc.subcore_barrier()
    pltpu.sync_copy(acc_sh.at[pl.ds(s * rpt, rpt)],
                    out_hbm.at[c, pl.ds(s * rpt, rpt)])


def _scale0_body(degt_ref, x_ref, y_ref):
    d = degt_ref[:, 0:1] + degt_ref[:, 1:2] + 1.0
    y_ref[...] = x_ref[...] * lax.rsqrt(d)


def _scale_mid_body(degt_ref, p_ref, y0_ref, y1_ref):
    d = degt_ref[:, 0:1] + degt_ref[:, 1:2] + 1.0
    a = p_ref[0, :, :] + p_ref[1, :, :] - y0_ref[...]
    y1_ref[...] = a / d


def _final_body(degt_ref, q_ref, y1_ref, w_ref, out_ref):
    d = degt_ref[:, 0:1] + degt_ref[:, 1:2] + 1.0
    h = (q_ref[0, :, :] + q_ref[1, :, :] - y1_ref[...]) * lax.rsqrt(d)
    out_ref[...] = lax.dot_general(h, w_ref[...], (((1,), (1,)), ((), ())),
                                   preferred_element_type=jnp.float32)


def kernel(x, edge_index, W):
    n, d = x.shape
    e = edge_index.shape[1]
    npad = NP
    epad = NW * CPT * CH
    assert epad >= e and npad >= n and npad % RB == 0 and (npad // NS) % 8 == 0
    xp = jnp.pad(x, ((0, npad - n), (0, 0)))
    pad_src = jnp.arange(epad - e, dtype=jnp.int32) % n
    pad_dst = n + jnp.arange(epad - e, dtype=jnp.int32) % (npad - n)
    src2 = jnp.concatenate([edge_index[0], pad_src]).reshape(epad // CH, CH)
    dst2 = jnp.concatenate([edge_index[1], pad_dst]).reshape(epad // CH, CH)
    mesh = plsc.VectorSubcoreMesh(core_axis_name="c", subcore_axis_name="s")

    deg = pl.kernel(
        _deg_body,
        out_type=jax.ShapeDtypeStruct((NC, 1, NPD), jnp.float32),
        mesh=mesh,
        scratch_types=[
            pltpu.VMEM_SHARED((NPD,), jnp.float32),
            pltpu.VMEM((CPTD, CHD), jnp.int32),
            pltpu.VMEM((NPD // NS,), jnp.float32),
            pltpu.VMEM((CHD,), jnp.float32),
        ],
    )(dst2.reshape(epad // CHD, CHD))
    degt = deg[:, 0, :npad].T

    hop = pl.kernel(
        _hop_body,
        out_type=jax.ShapeDtypeStruct((NC, npad, d), jnp.float32),
        mesh=mesh,
        scratch_types=[
            pltpu.VMEM_SHARED((npad, d), jnp.float32),
            pltpu.VMEM((CPT // SB, CH), jnp.int32),
            pltpu.VMEM((CPT // SB, CH), jnp.int32),
            pltpu.VMEM((CH, d), jnp.float32),
            pltpu.VMEM((CH, d), jnp.float32),
            pltpu.VMEM((CH, d), jnp.float32),
            pltpu.VMEM((CH, d), jnp.float32),
            pltpu.SemaphoreType.DMA,
            pltpu.SemaphoreType.DMA,
            pltpu.SemaphoreType.DMA,
            pltpu.SemaphoreType.DMA,
        ],
    )

    nb = npad // RB
    row = lambda i: (i, 0)
    full2 = pl.BlockSpec((RB, d), row)
    degb = pl.BlockSpec((RB, NC), row)
    pairb = pl.BlockSpec((NC, RB, d), lambda i: (0, i, 0))

    y0 = pl.pallas_call(
        _scale0_body,
        grid=(nb,),
        in_specs=[degb, full2],
        out_specs=full2,
        out_shape=jax.ShapeDtypeStruct((npad, d), jnp.float32),
    )(degt, xp)

    p = hop(y0, src2, dst2)

    y1 = pl.pallas_call(
        _scale_mid_body,
        grid=(nb,),
        in_specs=[degb, pairb, full2],
        out_specs=full2,
        out_shape=jax.ShapeDtypeStruct((npad, d), jnp.float32),
    )(degt, p, y0)

    q = hop(y1, src2, dst2)

    out = pl.pallas_call(
        _final_body,
        grid=(nb,),
        in_specs=[degb, pairb, full2,
                  pl.BlockSpec((d, d), lambda i: (0, 0))],
        out_specs=full2,
        out_shape=jax.ShapeDtypeStruct((npad, d), jnp.float32),
    )(degt, q, y1, W)
    return out[:n]


# speedup vs baseline: 34.0576x; 1.0858x over previous
import jax
import jax.numpy as jnp
from jax import lax
from jax.experimental import pallas as pl
from jax.experimental.pallas import tpu as pltpu
from jax.experimental.pallas import tpu_sc as plsc

NC = 2
NS = 16
NW = NC * NS
CH = 64
CPT = 160
SB = 4
CHD = 128
CPTD = 80
NP = 10112
NPD = 10240
RB = 1264


def _deg_body(dst2_hbm, deg_hbm, deg_sh, didx, zeros_v, ones_v):
    c = lax.axis_index("c")
    s = lax.axis_index("s")
    npad = deg_sh.shape[0]
    rpt = npad // NS
    for j in range((rpt + 15) // 16):
        zeros_v[pl.ds(min(j * 16, rpt - 16), 16)] = jnp.zeros((16,),
                                                              jnp.float32)
    for j in range(CHD // 16):
        ones_v[pl.ds(j * 16, 16)] = jnp.full((16,), 1.0, jnp.float32)
    pltpu.sync_copy(zeros_v, deg_sh.at[pl.ds(s * rpt, rpt)])
    rbase = (c * NS + s) * CPTD
    pltpu.sync_copy(dst2_hbm.at[pl.ds(rbase, CPTD)], didx)
    plsc.subcore_barrier()

    def chunk(i, carry):
        pltpu.sync_copy(ones_v, deg_sh.at[didx.at[i]], add=True)
        return carry

    lax.fori_loop(0, CPTD, chunk, 0)
    plsc.subcore_barrier()
    pltpu.sync_copy(deg_sh.at[pl.ds(s * rpt, rpt)],
                    deg_hbm.at[c, 0, pl.ds(s * rpt, rpt)])


def _hop_body(y_hbm, src2_hbm, dst2_hbm, out_hbm, acc_sh, sidx, didx,
              b0, b1, b2, b3, g0, g1, g2, g3):
    c = lax.axis_index("c")
    s = lax.axis_index("s")
    npad = y_hbm.shape[0]
    rpt = npad // NS
    pltpu.sync_copy(y_hbm.at[pl.ds(s * rpt, rpt)],
                    acc_sh.at[pl.ds(s * rpt, rpt)])
    rbase = (c * NS + s) * CPT
    plsc.subcore_barrier()

    bufs = (b0, b1, b2, b3)
    gsem = (g0, g1, g2, g3)

    def gather(i, k):
        pltpu.async_copy(y_hbm.at[sidx.at[i]], bufs[k], gsem[k])

    def gwait(k):
        pltpu.make_async_copy(y_hbm.at[sidx.at[0]], bufs[k], gsem[k]).wait()

    def scatter(i, k):
        pltpu.sync_copy(bufs[k], acc_sh.at[didx.at[i]], add=True)

    def step(i, k, do_gather=True):
        gwait(k)
        if do_gather:
            gather(i + 3, (k + 3) % 4)
        scatter(i, k)

    cpb = CPT // SB
    for h in range(SB):
        pltpu.sync_copy(src2_hbm.at[pl.ds(rbase + h * cpb, cpb)], sidx)
        pltpu.sync_copy(dst2_hbm.at[pl.ds(rbase + h * cpb, cpb)], didx)
        gather(0, 0)
        gather(1, 1)
        gather(2, 2)
        step(0, 0)
        step(1, 1)
        step(2, 2)
        step(3, 3)

        def quad(j, carry):
            i0 = 4 * j + 4
            step(i0, 0)
            step(i0 + 1, 1)
            step(i0 + 2, 2)
            step(i0 + 3, 3)
            return carry

        lax.fori_loop(0, (cpb - 8) // 4, quad, 0)
        step(cpb - 4, 0)
        step(cpb - 3, 1, do_gather=False)
        step(cpb - 2, 2, do_gather=False)
        step(cpb - 1, 3, do_gather=False)
    plsc.subcore_barrier()
    pltpu.sync_copy(acc_sh.at[pl.ds(s * rpt, rpt)],
                    out_hbm.at[c, pl.ds(s * rpt, rpt)])


def _scale0_body(degt_ref, x_ref, y_ref):
    d = degt_ref[:, 0:1] + degt_ref[:, 1:2] + 1.0
    y_ref[...] = x_ref[...] * lax.rsqrt(d)


def _scale_mid_body(degt_ref, p_ref, y0_ref, y1_ref):
    d = degt_ref[:, 0:1] + degt_ref[:, 1:2] + 1.0
    a = p_ref[0, :, :] + p_ref[1, :, :] - y0_ref[...]
    y1_ref[...] = a / d


def _final_body(degt_ref, q_ref, y1_ref, w_ref, out_ref):
    d = degt_ref[:, 0:1] + degt_ref[:, 1:2] + 1.0
    h = (q_ref[0, :, :] + q_ref[1, :, :] - y1_ref[...]) * lax.rsqrt(d)
    out_ref[...] = lax.dot_general(h, w_ref[...], (((1,), (1,)), ((), ())),
                                   preferred_element_type=jnp.float32)


def kernel(x, edge_index, W):
    n, d = x.shape
    e = edge_index.shape[1]
    npad = NP
    epad = NW * CPT * CH
    assert epad >= e and npad >= n and npad % RB == 0 and (npad // NS) % 8 == 0
    xp = jnp.pad(x, ((0, npad - n), (0, 0)))
    pad_src = jnp.arange(epad - e, dtype=jnp.int32) % n
    pad_dst = n + jnp.arange(epad - e, dtype=jnp.int32) % (npad - n)
    src2 = jnp.concatenate([edge_index[0], pad_src]).reshape(epad // CH, CH)
    dst2 = jnp.concatenate([edge_index[1], pad_dst]).reshape(epad // CH, CH)
    mesh = plsc.VectorSubcoreMesh(core_axis_name="c", subcore_axis_name="s")

    deg = pl.kernel(
        _deg_body,
        out_type=jax.ShapeDtypeStruct((NC, 1, NPD), jnp.float32),
        mesh=mesh,
        scratch_types=[
            pltpu.VMEM_SHARED((NPD,), jnp.float32),
            pltpu.VMEM((CPTD, CHD), jnp.int32),
            pltpu.VMEM((NPD // NS,), jnp.float32),
            pltpu.VMEM((CHD,), jnp.float32),
        ],
    )(dst2.reshape(epad // CHD, CHD))
    degt = deg[:, 0, :npad].T

    hop = pl.kernel(
        _hop_body,
        out_type=jax.ShapeDtypeStruct((NC, npad, d), jnp.float32),
        mesh=mesh,
        scratch_types=[
            pltpu.VMEM_SHARED((npad, d), jnp.float32),
            pltpu.VMEM((CPT // SB, CH), jnp.int32),
            pltpu.VMEM((CPT // SB, CH), jnp.int32),
            pltpu.VMEM((CH, d), jnp.float32),
            pltpu.VMEM((CH, d), jnp.float32),
            pltpu.VMEM((CH, d), jnp.float32),
            pltpu.VMEM((CH, d), jnp.float32),
            pltpu.SemaphoreType.DMA,
            pltpu.SemaphoreType.DMA,
            pltpu.SemaphoreType.DMA,
            pltpu.SemaphoreType.DMA,
        ],
    )

    nb = npad // RB
    row = lambda i: (i, 0)
    full2 = pl.BlockSpec((RB, d), row)
    degb = pl.BlockSpec((RB, NC), row)
    pairb = pl.BlockSpec((NC, RB, d), lambda i: (0, i, 0))

    y0 = pl.pallas_call(
        _scale0_body,
        grid=(nb,),
        in_specs=[degb, full2],
        out_specs=full2,
        out_shape=jax.ShapeDtypeStruct((npad, d), jnp.float32),
    )(degt, xp)

    p = hop(y0, src2, dst2)

    y1 = pl.pallas_call(
        _scale_mid_body,
        grid=(nb,),
        in_specs=[degb, pairb, full2],
        out_specs=full2,
        out_shape=jax.ShapeDtypeStruct((npad, d), jnp.float32),
    )(degt, p, y0)

    q = hop(y1, src2, dst2)

    out = pl.pallas_call(
        _final_body,
        grid=(nb,),
        in_specs=[degb, pairb, full2,
                  pl.BlockSpec((d, d), lambda i: (0, 0))],
        out_specs=full2,
        out_shape=jax.ShapeDtypeStruct((npad, d), jnp.float32),
    )(degt, q, y1, W)
    return out[:n]


# speedup vs baseline: 34.1898x; 1.0039x over previous
import jax
import jax.numpy as jnp
from jax import lax
from jax.experimental import pallas as pl
from jax.experimental.pallas import tpu as pltpu
from jax.experimental.pallas import tpu_sc as plsc

NC = 2
NS = 16
NW = NC * NS
CH = 64
CPT = 160
SB = 4
CHD = 128
CPTD = 80
NP = 10112
NPD = 10240
RB = 1264


def _deg_body(dst2_hbm, deg_hbm, deg_sh, didx, zeros_v, ones_v):
    c = lax.axis_index("c")
    s = lax.axis_index("s")
    npad = deg_sh.shape[0]
    rpt = npad // NS
    for j in range((rpt + 15) // 16):
        zeros_v[pl.ds(min(j * 16, rpt - 16), 16)] = jnp.zeros((16,),
                                                              jnp.float32)
    for j in range(CHD // 16):
        ones_v[pl.ds(j * 16, 16)] = jnp.full((16,), 1.0, jnp.float32)
    pltpu.sync_copy(zeros_v, deg_sh.at[pl.ds(s * rpt, rpt)])
    rbase = (c * NS + s) * CPTD
    pltpu.sync_copy(dst2_hbm.at[pl.ds(rbase, CPTD)], didx)
    plsc.subcore_barrier()

    def chunk(i, carry):
        pltpu.sync_copy(ones_v, deg_sh.at[didx.at[i]], add=True)
        return carry

    lax.fori_loop(0, CPTD, chunk, 0)
    plsc.subcore_barrier()
    pltpu.sync_copy(deg_sh.at[pl.ds(s * rpt, rpt)],
                    deg_hbm.at[c, 0, pl.ds(s * rpt, rpt)])


def _hop_body(y_hbm, src2_hbm, dst2_hbm, out_hbm, acc_sh, sidx, didx,
              b0, b1, b2, b3, g0, g1, g2, g3):
    c = lax.axis_index("c")
    s = lax.axis_index("s")
    npad = y_hbm.shape[0]
    rpt = npad // NS
    ch = b0.shape[0]
    rbase = (c * NS + s) * CPT

    @pl.when(c == 0)
    def _():
        pltpu.sync_copy(y_hbm.at[pl.ds(s * rpt, rpt)],
                        acc_sh.at[pl.ds(s * rpt, rpt)])

    @pl.when(c == 1)
    def _():
        def zrow(r, carry):
            for j in range(b0.shape[1] // 16):
                b0[r, pl.ds(j * 16, 16)] = jnp.zeros((16,), jnp.float32)
            return carry

        lax.fori_loop(0, ch, zrow, 0)
        for k in range(rpt // ch):
            pltpu.sync_copy(b0, acc_sh.at[pl.ds(s * rpt + k * ch, ch)])
        rem = rpt % ch
        if rem:
            pltpu.sync_copy(b0.at[pl.ds(0, rem)],
                            acc_sh.at[pl.ds(s * rpt + rpt - rem, rem)])

    plsc.subcore_barrier()

    bufs = (b0, b1, b2, b3)
    gsem = (g0, g1, g2, g3)

    def gather(i, k):
        pltpu.async_copy(y_hbm.at[sidx.at[i]], bufs[k], gsem[k])

    def gwait(k):
        pltpu.make_async_copy(y_hbm.at[sidx.at[0]], bufs[k], gsem[k]).wait()

    def scatter(i, k):
        pltpu.sync_copy(bufs[k], acc_sh.at[didx.at[i]], add=True)

    def step(i, k, do_gather=True):
        gwait(k)
        if do_gather:
            gather(i + 3, (k + 3) % 4)
        scatter(i, k)

    cpb = CPT // SB
    for h in range(SB):
        pltpu.sync_copy(src2_hbm.at[pl.ds(rbase + h * cpb, cpb)], sidx)
        pltpu.sync_copy(dst2_hbm.at[pl.ds(rbase + h * cpb, cpb)], didx)
        gather(0, 0)
        gather(1, 1)
        gather(2, 2)
        step(0, 0)
        step(1, 1)
        step(2, 2)
        step(3, 3)

        def quad(j, carry):
            i0 = 4 * j + 4
            step(i0, 0)
            step(i0 + 1, 1)
            step(i0 + 2, 2)
            step(i0 + 3, 3)
            return carry

        lax.fori_loop(0, (cpb - 8) // 4, quad, 0)
        step(cpb - 4, 0)
        step(cpb - 3, 1, do_gather=False)
        step(cpb - 2, 2, do_gather=False)
        step(cpb - 1, 3, do_gather=False)
    plsc.subcore_barrier()
    pltpu.sync_copy(acc_sh.at[pl.ds(s * rpt, rpt)],
                    out_hbm.at[c, pl.ds(s * rpt, rpt)])


def _scale0_body(degt_ref, x_ref, y_ref):
    d = degt_ref[:, 0:1] + degt_ref[:, 1:2] + 1.0
    y_ref[...] = x_ref[...] * lax.rsqrt(d)


def _scale_mid_body(degt_ref, p_ref, y1_ref):
    d = degt_ref[:, 0:1] + degt_ref[:, 1:2] + 1.0
    y1_ref[...] = (p_ref[0, :, :] + p_ref[1, :, :]) / d


def _final_body(degt_ref, q_ref, w_ref, out_ref):
    d = degt_ref[:, 0:1] + degt_ref[:, 1:2] + 1.0
    h = (q_ref[0, :, :] + q_ref[1, :, :]) * lax.rsqrt(d)
    out_ref[...] = lax.dot_general(h, w_ref[...], (((1,), (1,)), ((), ())),
                                   preferred_element_type=jnp.float32)


def kernel(x, edge_index, W):
    n, d = x.shape
    e = edge_index.shape[1]
    npad = NP
    epad = NW * CPT * CH
    assert epad >= e and npad >= n and npad % RB == 0 and (npad // NS) % 8 == 0
    xp = jnp.pad(x, ((0, npad - n), (0, 0)))
    pad_src = jnp.arange(epad - e, dtype=jnp.int32) % n
    pad_dst = n + jnp.arange(epad - e, dtype=jnp.int32) % (npad - n)
    src2 = jnp.concatenate([edge_index[0], pad_src]).reshape(epad // CH, CH)
    dst2 = jnp.concatenate([edge_index[1], pad_dst]).reshape(epad // CH, CH)
    mesh = plsc.VectorSubcoreMesh(core_axis_name="c", subcore_axis_name="s")

    deg = pl.kernel(
        _deg_body,
        out_type=jax.ShapeDtypeStruct((NC, 1, NPD), jnp.float32),
        mesh=mesh,
        scratch_types=[
            pltpu.VMEM_SHARED((NPD,), jnp.float32),
            pltpu.VMEM((CPTD, CHD), jnp.int32),
            pltpu.VMEM((NPD // NS,), jnp.float32),
            pltpu.VMEM((CHD,), jnp.float32),
        ],
    )(dst2.reshape(epad // CHD, CHD))
    degt = deg[:, 0, :npad].T

    hop = pl.kernel(
        _hop_body,
        out_type=jax.ShapeDtypeStruct((NC, npad, d), jnp.float32),
        mesh=mesh,
        scratch_types=[
            pltpu.VMEM_SHARED((npad, d), jnp.float32),
            pltpu.VMEM((CPT // SB, CH), jnp.int32),
            pltpu.VMEM((CPT // SB, CH), jnp.int32),
            pltpu.VMEM((CH, d), jnp.float32),
            pltpu.VMEM((CH, d), jnp.float32),
            pltpu.VMEM((CH, d), jnp.float32),
            pltpu.VMEM((CH, d), jnp.float32),
            pltpu.SemaphoreType.DMA,
            pltpu.SemaphoreType.DMA,
            pltpu.SemaphoreType.DMA,
            pltpu.SemaphoreType.DMA,
        ],
    )

    nb = npad // RB
    row = lambda i: (i, 0)
    full2 = pl.BlockSpec((RB, d), row)
    degb = pl.BlockSpec((RB, NC), row)
    pairb = pl.BlockSpec((NC, RB, d), lambda i: (0, i, 0))

    y0 = pl.pallas_call(
        _scale0_body,
        grid=(nb,),
        in_specs=[degb, full2],
        out_specs=full2,
        out_shape=jax.ShapeDtypeStruct((npad, d), jnp.float32),
    )(degt, xp)

    p = hop(y0, src2, dst2)

    y1 = pl.pallas_call(
        _scale_mid_body,
        grid=(nb,),
        in_specs=[degb, pairb],
        out_specs=full2,
        out_shape=jax.ShapeDtypeStruct((npad, d), jnp.float32),
    )(degt, p)

    q = hop(y1, src2, dst2)

    out = pl.pallas_call(
        _final_body,
        grid=(nb,),
        in_specs=[degb, pairb,
                  pl.BlockSpec((d, d), lambda i: (0, 0))],
        out_specs=full2,
        out_shape=jax.ShapeDtypeStruct((npad, d), jnp.float32),
    )(degt, q, W)
    return out[:n]
